# Initial kernel scaffold; baseline (speedup 1.0000x reference)
import jax
import jax.numpy as jnp
from jax.experimental import pallas as pl


def kernel(c_input, weight):
    raise NotImplementedError("write your pallas kernel here")


# speedup vs baseline: 5.5952x; 5.5952x over previous
import jax
import jax.numpy as jnp
from jax.experimental import pallas as pl
from jax.experimental.pallas import tpu as pltpu

N = 325
TOPK = 16


def _body(x_ref, w_ref, o_ref):
    x = x_ref[0]
    w0 = w_ref[0]
    w1 = w_ref[1]
    a1 = jnp.dot(x, w0, preferred_element_type=jnp.float32)
    a2 = jnp.dot(x, w1, preferred_element_type=jnp.float32)
    g = jax.lax.dot_general(a1, a2, (((1,), (1,)), ((), ())),
                            preferred_element_type=jnp.float32)
    g = jnp.maximum(g, 0.0)

    cur = g
    remaining = jnp.full((N, 1), TOPK, dtype=jnp.float32)
    thresh = jnp.zeros((N, 1), dtype=jnp.float32)
    done = jnp.zeros((N, 1), dtype=jnp.bool_)
    row_max = jnp.max(g, axis=1, keepdims=True)
    for _ in range(TOPK):
        m = jnp.max(cur, axis=1, keepdims=True)
        is_m = cur == m
        c = jnp.sum(is_m.astype(jnp.float32), axis=1, keepdims=True)
        done_now = jnp.logical_and(c >= remaining, jnp.logical_not(done))
        thresh = jnp.where(done_now, m, thresh)
        done = jnp.logical_or(done, done_now)
        remaining = remaining - jnp.where(done, 0.0, c)
        cur = jnp.where(is_m, -1.0, cur)

    data = jnp.where(g < thresh, 0.0, g)
    mask = data != 0.0
    e = jnp.where(mask, jnp.exp(data - row_max), 0.0)
    s = jnp.sum(e, axis=1, keepdims=True) + 1e-5
    o_ref[0] = e / s


def kernel(c_input, weight):
    b, t, n, d = c_input.shape
    xs = c_input.reshape(b * t, n, d)
    out = pl.pallas_call(
        _body,
        grid=(b * t,),
        in_specs=[
            pl.BlockSpec((1, n, d), lambda i: (i, 0, 0)),
            pl.BlockSpec((2, d, weight.shape[2]), lambda i: (0, 0, 0)),
        ],
        out_specs=pl.BlockSpec((1, n, n), lambda i: (i, 0, 0)),
        out_shape=jax.ShapeDtypeStruct((b * t, n, n), jnp.float32),
    )(xs, weight)
    return out.reshape(b, t, n, n)
